# baseline (device time: 151114 ns/iter reference)
import functools

import jax
import jax.numpy as jnp
from jax import lax
from jax.experimental import pallas as pl
from jax.experimental.pallas import tpu as pltpu

N_DEV = 4


def kernel(x, w_mat):
    m, k_per = x.shape
    _, n = w_mat.shape
    ch = m // N_DEV
    nh = n // 2

    w_mat = w_mat.astype(jnp.bfloat16)

    def body(x_ref, w_ref, out_ref, acc_cw, acc_ccw, q_ref,
             deq_tiles, amax_ref, send_cw, recv_cw, send_ccw,
             recv_ccw, amax_send_sems, amax_recv_sems, deq_sems,
             credit_cw, credit_ccw):
        my = lax.axis_index("i")
        left = lax.rem(my + N_DEV - 1, N_DEV)
        right = lax.rem(my + 1, N_DEV)

        barrier_sem = pltpu.get_barrier_semaphore()
        for nbr in (left, right):
            pl.semaphore_signal(
                barrier_sem, inc=1,
                device_id=(nbr,), device_id_type=pl.DeviceIdType.MESH,
            )
        pl.semaphore_wait(barrier_sem, 2)

        SUB = 256
        HALF = ch // 2
        QTR = ch // 4
        N_RS_SUB = 4

        def gemm_tile(c, col0, r0):
            return jnp.dot(
                x_ref[pl.ds(c * ch + r0, SUB), :].astype(jnp.bfloat16),
                w_ref[:, pl.ds(col0, nh)],
                preferred_element_type=jnp.float32,
            )

        own_maxes = []

        def fused_add(acc, slot, c, col0, r_lo, r_hi, track):
            for r0 in range(r_lo, r_hi, SUB):
                sl = pl.ds(r0, SUB)
                v = gemm_tile(c, col0, r0) + acc[slot, sl, :].astype(
                    jnp.float32
                )
                acc[slot, sl, :] = v.astype(jnp.bfloat16)
                if track:
                    own_maxes.append(jnp.max(jnp.abs(v)))

        def rs_rdma(acc, src_slot, dst_slot, s_sems, r_sems, s, sub,
                    tgt):
            rows = pl.ds(sub * QTR, QTR)
            return pltpu.make_async_remote_copy(
                src_ref=acc.at[src_slot, rows, :],
                dst_ref=acc.at[dst_slot, rows, :],
                send_sem=s_sems.at[N_RS_SUB * s + sub],
                recv_sem=r_sems.at[N_RS_SUB * s + sub],
                device_id=(tgt,),
                device_id_type=pl.DeviceIdType.MESH,
            )

        c_own_cw = lax.rem(my + 1, N_DEV)
        c_own_ccw = lax.rem(my + N_DEV - 1, N_DEV)

        pending = {}
        for sub in range(N_RS_SUB):
            for d, acc, s_sems, r_sems, col0, tgt in (
                ("cw", acc_cw, send_cw, recv_cw, 0, right),
                ("ccw", acc_ccw, send_ccw, recv_ccw, nh, left),
            ):
                for r0 in range(sub * QTR, (sub + 1) * QTR, SUB):
                    acc[1, pl.ds(r0, SUB), :] = gemm_tile(
                        my, col0, r0
                    ).astype(jnp.bfloat16)
                r = rs_rdma(acc, 1, 0, s_sems, r_sems, 0, sub, tgt)
                r.start()
                pending[d, sub] = r

        for s in range(N_DEV - 1):
            src_slot = (s + 1) % 2
            dst_slot = s % 2
            if s >= 1:
                pl.semaphore_wait(credit_cw, 1)
                pl.semaphore_wait(credit_ccw, 1)
                for sub in range(N_RS_SUB):
                    r = rs_rdma(acc_cw, src_slot, dst_slot, send_cw,
                                recv_cw, s, sub, right)
                    r.start()
                    pending["cw", sub] = r
                    r = rs_rdma(acc_ccw, src_slot, dst_slot, send_ccw,
                                recv_ccw, s, sub, left)
                    r.start()
                    pending["ccw", sub] = r
            c_recv_cw = lax.rem(my - s - 1 + N_DEV, N_DEV)
            c_recv_ccw = lax.rem(my + s + 1, N_DEV)
            track = s == N_DEV - 2
            for sub in range(N_RS_SUB):
                r_lo, r_hi = sub * QTR, (sub + 1) * QTR
                pending["cw", sub].wait()
                fused_add(acc_cw, dst_slot, c_recv_cw, 0, r_lo, r_hi,
                          track)
                pending["ccw", sub].wait()
                fused_add(acc_ccw, dst_slot, c_recv_ccw, nh, r_lo, r_hi,
                          track)
            if s < N_DEV - 2:
                pl.semaphore_signal(
                    credit_cw, inc=1,
                    device_id=(left,), device_id_type=pl.DeviceIdType.MESH,
                )
                pl.semaphore_signal(
                    credit_ccw, inc=1,
                    device_id=(right,), device_id_type=pl.DeviceIdType.MESH,
                )

        amax_loc = functools.reduce(jnp.maximum, own_maxes)

        amax_ref[my] = jnp.full((8, 128), amax_loc, dtype=jnp.float32)
        amax_rdmas = []
        for t in range(N_DEV - 1):
            tgt = lax.rem(my + 1 + t, N_DEV)
            rdma = pltpu.make_async_remote_copy(
                src_ref=amax_ref.at[my],
                dst_ref=amax_ref.at[my],
                send_sem=amax_send_sems.at[t],
                recv_sem=amax_recv_sems.at[t],
                device_id=(tgt,),
                device_id_type=pl.DeviceIdType.MESH,
            )
            rdma.start()
            amax_rdmas.append(rdma)
        for rdma in amax_rdmas:
            rdma.wait()
        amax = jnp.max(amax_ref[...])

        scale = amax / 127.0
        inv_scale = 1.0 / scale

        def quant_rows(acc, c_own, col0, r_lo, r_hi):
            for r0 in range(r_lo, r_hi, SUB):
                yq = jnp.clip(
                    jnp.round(
                        acc[0, pl.ds(r0, SUB), :].astype(jnp.float32)
                        * inv_scale
                    ),
                    -127.0, 127.0,
                )
                q_ref[pl.ds(c_own * ch + r0, SUB), pl.ds(col0, nh)] = (
                    yq.astype(jnp.int8)
                )

        deq_pending = {}

        def dequant_rows(c, col0, r_lo, r_hi):
            for r0 in range(r_lo, r_hi, SUB):
                slot = (r0 // SUB) % 2
                if slot in deq_pending:
                    deq_pending.pop(slot).wait()
                sl = pl.ds(c * ch + r0, SUB)
                cs = pl.ds(col0, nh)
                deq_tiles[slot] = (
                    q_ref[sl, cs].astype(jnp.float32) * scale
                ).astype(jnp.bfloat16)
                copy = pltpu.make_async_copy(
                    deq_tiles.at[slot],
                    out_ref.at[sl, cs],
                    deq_sems.at[slot],
                )
                copy.start()
                deq_pending[slot] = copy

        def ag_rdma(c, col0, s_sems, r_sems, t, sub, tgt):
            rows = pl.ds(c * ch + sub * HALF, HALF)
            return pltpu.make_async_remote_copy(
                src_ref=q_ref.at[rows, pl.ds(col0, nh)],
                dst_ref=q_ref.at[rows, pl.ds(col0, nh)],
                send_sem=s_sems.at[4 * (N_DEV - 1) + 2 * t + sub],
                recv_sem=r_sems.at[4 * (N_DEV - 1) + 2 * t + sub],
                device_id=(tgt,),
                device_id_type=pl.DeviceIdType.MESH,
            )

        for t in range(N_DEV - 1):
            c_cw = lax.rem(my + 1 - t + N_DEV, N_DEV)
            c_ccw = lax.rem(my - 1 + t + N_DEV, N_DEV)
            ag_pending = []
            for sub in (0, 1):
                if t == 0:
                    quant_rows(acc_cw, c_own_cw, 0,
                               sub * HALF, (sub + 1) * HALF)
                r = ag_rdma(c_cw, 0, send_cw, recv_cw, t, sub, right)
                r.start()
                ag_pending.append(r)
                if t == 0:
                    quant_rows(acc_ccw, c_own_ccw, nh,
                               sub * HALF, (sub + 1) * HALF)
                r = ag_rdma(c_ccw, nh, send_ccw, recv_ccw, t, sub, left)
                r.start()
                ag_pending.append(r)
            dequant_rows(c_cw, 0, 0, ch)
            dequant_rows(c_ccw, nh, 0, ch)
            if t < N_DEV - 2:
                for r in ag_pending:
                    r.wait()
            else:
                c_last = lax.rem(my + 2, N_DEV)
                ag_pending[0].wait()
                ag_pending[1].wait()
                dequant_rows(c_last, 0, 0, HALF)
                dequant_rows(c_last, nh, 0, HALF)
                ag_pending[2].wait()
                ag_pending[3].wait()
                dequant_rows(c_last, 0, HALF, ch)
                dequant_rows(c_last, nh, HALF, ch)

        for copy in deq_pending.values():
            copy.wait()

    n_sems = 4 * (N_DEV - 1) + 2 * (N_DEV - 1)
    return pl.pallas_call(
        body,
        out_shape=jax.ShapeDtypeStruct((m, n), jnp.bfloat16),
        in_specs=[
            pl.BlockSpec(memory_space=pltpu.VMEM),
            pl.BlockSpec(memory_space=pltpu.VMEM),
        ],
        out_specs=pl.BlockSpec(memory_space=pl.ANY),
        scratch_shapes=[
            pltpu.VMEM((2, ch, n // 2), jnp.bfloat16),
            pltpu.VMEM((2, ch, n // 2), jnp.bfloat16),
            pltpu.VMEM((m, n), jnp.int8),
            pltpu.VMEM((2, 256, n // 2), jnp.bfloat16),
            pltpu.VMEM((N_DEV, 8, 128), jnp.float32),
            pltpu.SemaphoreType.DMA((n_sems,)),
            pltpu.SemaphoreType.DMA((n_sems,)),
            pltpu.SemaphoreType.DMA((n_sems,)),
            pltpu.SemaphoreType.DMA((n_sems,)),
            pltpu.SemaphoreType.DMA((N_DEV - 1,)),
            pltpu.SemaphoreType.DMA((N_DEV - 1,)),
            pltpu.SemaphoreType.DMA((2,)),
            pltpu.SemaphoreType.REGULAR,
            pltpu.SemaphoreType.REGULAR,
        ],
        compiler_params=pltpu.CompilerParams(
            collective_id=0,
            vmem_limit_bytes=40 * 1024 * 1024,
        ),
    )(x, w_mat)


# device time: 147676 ns/iter; 1.0233x vs baseline; 1.0233x over previous
import functools

import jax
import jax.numpy as jnp
from jax import lax
from jax.experimental import pallas as pl
from jax.experimental.pallas import tpu as pltpu

N_DEV = 4


def kernel(x, w_mat):
    m, k_per = x.shape
    _, n = w_mat.shape
    ch = m // N_DEV
    nh = n // 2

    w_mat = w_mat.astype(jnp.bfloat16)

    def body(x_ref, w_ref, out_ref, acc_cw, acc_ccw, q_ref, stage_cw,
             stage_ccw, deq_tiles, amax_ref, send_cw, recv_cw, send_ccw,
             recv_ccw, amax_send_sems, amax_recv_sems, deq_sems,
             credit_cw, credit_ccw):
        my = lax.axis_index("i")
        left = lax.rem(my + N_DEV - 1, N_DEV)
        right = lax.rem(my + 1, N_DEV)

        barrier_sem = pltpu.get_barrier_semaphore()
        for nbr in (left, right):
            pl.semaphore_signal(
                barrier_sem, inc=1,
                device_id=(nbr,), device_id_type=pl.DeviceIdType.MESH,
            )
        pl.semaphore_wait(barrier_sem, 2)

        SUB = 256
        HALF = ch // 2
        QTR = ch // 4
        N_RS_SUB = 4

        def gemm_into(dst, c, col0, r_lo, r_hi):
            for r0 in range(r_lo, r_hi, SUB):
                p = jnp.dot(
                    x_ref[pl.ds(c * ch + r0, SUB), :].astype(jnp.bfloat16),
                    w_ref[:, pl.ds(col0, nh)],
                    preferred_element_type=jnp.float32,
                )
                dst[pl.ds(r0, SUB), :] = p.astype(jnp.bfloat16)

        own_maxes = []

        def add_stage(acc, slot, stage, r_lo, r_hi, track):
            for r0 in range(r_lo, r_hi, SUB):
                sl = pl.ds(r0, SUB)
                v = (
                    acc[slot, sl, :].astype(jnp.float32)
                    + stage[sl, :].astype(jnp.float32)
                )
                acc[slot, sl, :] = v.astype(jnp.bfloat16)
                if track:
                    own_maxes.append(jnp.max(jnp.abs(v)))

        def rs_rdma(acc, src_slot, dst_slot, s_sems, r_sems, s, sub,
                    tgt):
            rows = pl.ds(sub * QTR, QTR)
            return pltpu.make_async_remote_copy(
                src_ref=acc.at[src_slot, rows, :],
                dst_ref=acc.at[dst_slot, rows, :],
                send_sem=s_sems.at[N_RS_SUB * s + sub],
                recv_sem=r_sems.at[N_RS_SUB * s + sub],
                device_id=(tgt,),
                device_id_type=pl.DeviceIdType.MESH,
            )

        c_own_cw = lax.rem(my + 1, N_DEV)
        c_own_ccw = lax.rem(my + N_DEV - 1, N_DEV)

        pending = {}
        for sub in range(N_RS_SUB):
            r_lo, r_hi = sub * QTR, (sub + 1) * QTR
            gemm_into(acc_cw.at[1], my, 0, r_lo, r_hi)
            r = rs_rdma(acc_cw, 1, 0, send_cw, recv_cw, 0, sub, right)
            r.start()
            pending["cw", sub] = r
            gemm_into(acc_ccw.at[1], my, nh, r_lo, r_hi)
            r = rs_rdma(acc_ccw, 1, 0, send_ccw, recv_ccw, 0, sub, left)
            r.start()
            pending["ccw", sub] = r

        for s in range(N_DEV - 1):
            src_slot = (s + 1) % 2
            dst_slot = s % 2
            if s >= 1:
                pl.semaphore_wait(credit_cw, 1)
                pl.semaphore_wait(credit_ccw, 1)
                for sub in range(N_RS_SUB):
                    r = rs_rdma(acc_cw, src_slot, dst_slot, send_cw,
                                recv_cw, s, sub, right)
                    r.start()
                    pending["cw", sub] = r
                    r = rs_rdma(acc_ccw, src_slot, dst_slot, send_ccw,
                                recv_ccw, s, sub, left)
                    r.start()
                    pending["ccw", sub] = r
            c_recv_cw = lax.rem(my - s - 1 + N_DEV, N_DEV)
            c_recv_ccw = lax.rem(my + s + 1, N_DEV)
            gemm_into(stage_cw, c_recv_cw, 0, 0, ch)
            gemm_into(stage_ccw, c_recv_ccw, nh, 0, ch)
            track = s == N_DEV - 2
            for sub in range(N_RS_SUB):
                r_lo, r_hi = sub * QTR, (sub + 1) * QTR
                pending["cw", sub].wait()
                if sub == N_RS_SUB - 1 and s < N_DEV - 2:
                    pl.semaphore_signal(
                        credit_cw, inc=1,
                        device_id=(left,),
                        device_id_type=pl.DeviceIdType.MESH,
                    )
                add_stage(acc_cw, dst_slot, stage_cw, r_lo, r_hi, track)
                pending["ccw", sub].wait()
                if sub == N_RS_SUB - 1 and s < N_DEV - 2:
                    pl.semaphore_signal(
                        credit_ccw, inc=1,
                        device_id=(right,),
                        device_id_type=pl.DeviceIdType.MESH,
                    )
                add_stage(acc_ccw, dst_slot, stage_ccw, r_lo, r_hi,
                          track)

        amax_loc = functools.reduce(jnp.maximum, own_maxes)

        amax_ref[my] = jnp.full((8, 128), amax_loc, dtype=jnp.float32)
        amax_rdmas = []
        for t in range(N_DEV - 1):
            tgt = lax.rem(my + 1 + t, N_DEV)
            rdma = pltpu.make_async_remote_copy(
                src_ref=amax_ref.at[my],
                dst_ref=amax_ref.at[my],
                send_sem=amax_send_sems.at[t],
                recv_sem=amax_recv_sems.at[t],
                device_id=(tgt,),
                device_id_type=pl.DeviceIdType.MESH,
            )
            rdma.start()
            amax_rdmas.append(rdma)
        for rdma in amax_rdmas:
            rdma.wait()
        amax = jnp.max(amax_ref[...])

        scale = amax / 127.0
        inv_scale = 1.0 / scale

        def quant_rows(acc, c_own, col0, r_lo, r_hi):
            for r0 in range(r_lo, r_hi, SUB):
                yq = jnp.clip(
                    jnp.round(
                        acc[0, pl.ds(r0, SUB), :].astype(jnp.float32)
                        * inv_scale
                    ),
                    -127.0, 127.0,
                )
                q_ref[pl.ds(c_own * ch + r0, SUB), pl.ds(col0, nh)] = (
                    yq.astype(jnp.int8)
                )

        deq_pending = {}

        def dequant_rows(c, col0, r_lo, r_hi):
            for r0 in range(r_lo, r_hi, SUB):
                slot = (r0 // SUB) % 2
                if slot in deq_pending:
                    deq_pending.pop(slot).wait()
                sl = pl.ds(c * ch + r0, SUB)
                cs = pl.ds(col0, nh)
                deq_tiles[slot] = (
                    q_ref[sl, cs].astype(jnp.float32) * scale
                ).astype(jnp.bfloat16)
                copy = pltpu.make_async_copy(
                    deq_tiles.at[slot],
                    out_ref.at[sl, cs],
                    deq_sems.at[slot],
                )
                copy.start()
                deq_pending[slot] = copy

        def ag_rdma(c, col0, s_sems, r_sems, t, sub, tgt):
            rows = pl.ds(c * ch + sub * HALF, HALF)
            return pltpu.make_async_remote_copy(
                src_ref=q_ref.at[rows, pl.ds(col0, nh)],
                dst_ref=q_ref.at[rows, pl.ds(col0, nh)],
                send_sem=s_sems.at[4 * (N_DEV - 1) + 2 * t + sub],
                recv_sem=r_sems.at[4 * (N_DEV - 1) + 2 * t + sub],
                device_id=(tgt,),
                device_id_type=pl.DeviceIdType.MESH,
            )

        for t in range(N_DEV - 1):
            c_cw = lax.rem(my + 1 - t + N_DEV, N_DEV)
            c_ccw = lax.rem(my - 1 + t + N_DEV, N_DEV)
            ag_pending = []
            for sub in (0, 1):
                if t == 0:
                    quant_rows(acc_cw, c_own_cw, 0,
                               sub * HALF, (sub + 1) * HALF)
                r = ag_rdma(c_cw, 0, send_cw, recv_cw, t, sub, right)
                r.start()
                ag_pending.append(r)
                if t == 0:
                    quant_rows(acc_ccw, c_own_ccw, nh,
                               sub * HALF, (sub + 1) * HALF)
                r = ag_rdma(c_ccw, nh, send_ccw, recv_ccw, t, sub, left)
                r.start()
                ag_pending.append(r)
            dequant_rows(c_cw, 0, 0, ch)
            dequant_rows(c_ccw, nh, 0, ch)
            if t < N_DEV - 2:
                for r in ag_pending:
                    r.wait()
            else:
                c_last = lax.rem(my + 2, N_DEV)
                ag_pending[0].wait()
                ag_pending[1].wait()
                dequant_rows(c_last, 0, 0, HALF)
                dequant_rows(c_last, nh, 0, HALF)
                ag_pending[2].wait()
                ag_pending[3].wait()
                dequant_rows(c_last, 0, HALF, ch)
                dequant_rows(c_last, nh, HALF, ch)

        for copy in deq_pending.values():
            copy.wait()

    n_sems = 4 * (N_DEV - 1) + 2 * (N_DEV - 1)
    return pl.pallas_call(
        body,
        out_shape=jax.ShapeDtypeStruct((m, n), jnp.bfloat16),
        in_specs=[
            pl.BlockSpec(memory_space=pltpu.VMEM),
            pl.BlockSpec(memory_space=pltpu.VMEM),
        ],
        out_specs=pl.BlockSpec(memory_space=pl.ANY),
        scratch_shapes=[
            pltpu.VMEM((2, ch, n // 2), jnp.bfloat16),
            pltpu.VMEM((2, ch, n // 2), jnp.bfloat16),
            pltpu.VMEM((m, n), jnp.int8),
            pltpu.VMEM((ch, n // 2), jnp.bfloat16),
            pltpu.VMEM((ch, n // 2), jnp.bfloat16),
            pltpu.VMEM((2, 256, n // 2), jnp.bfloat16),
            pltpu.VMEM((N_DEV, 8, 128), jnp.float32),
            pltpu.SemaphoreType.DMA((n_sems,)),
            pltpu.SemaphoreType.DMA((n_sems,)),
            pltpu.SemaphoreType.DMA((n_sems,)),
            pltpu.SemaphoreType.DMA((n_sems,)),
            pltpu.SemaphoreType.DMA((N_DEV - 1,)),
            pltpu.SemaphoreType.DMA((N_DEV - 1,)),
            pltpu.SemaphoreType.DMA((2,)),
            pltpu.SemaphoreType.REGULAR,
            pltpu.SemaphoreType.REGULAR,
        ],
        compiler_params=pltpu.CompilerParams(
            collective_id=0,
            vmem_limit_bytes=40 * 1024 * 1024,
        ),
    )(x, w_mat)


# device time: 139136 ns/iter; 1.0861x vs baseline; 1.0614x over previous
import functools

import jax
import jax.numpy as jnp
from jax import lax
from jax.experimental import pallas as pl
from jax.experimental.pallas import tpu as pltpu

N_DEV = 4


def kernel(x, w_mat):
    m, k_per = x.shape
    _, n = w_mat.shape
    ch = m // N_DEV
    nh = n // 2

    w_mat = w_mat.astype(jnp.bfloat16)

    def body(x_ref, w_ref, out_ref, acc_cw, acc_ccw, q_ref, stage_cw,
             stage_ccw, deq_tiles, amax_ref, send_cw, recv_cw, send_ccw,
             recv_ccw, amax_send_sems, amax_recv_sems, deq_sems,
             credit_cw, credit_ccw):
        my = lax.axis_index("i")
        left = lax.rem(my + N_DEV - 1, N_DEV)
        right = lax.rem(my + 1, N_DEV)

        barrier_sem = pltpu.get_barrier_semaphore()
        for nbr in (left, right):
            pl.semaphore_signal(
                barrier_sem, inc=1,
                device_id=(nbr,), device_id_type=pl.DeviceIdType.MESH,
            )
        pl.semaphore_wait(barrier_sem, 2)

        SUB = 256
        HALF = ch // 2
        QTR = ch // 4
        N_RS_SUB = 4

        def gemm_into(dst, c, col0, r_lo, r_hi):
            for r0 in range(r_lo, r_hi, SUB):
                p = jnp.dot(
                    x_ref[pl.ds(c * ch + r0, SUB), :].astype(jnp.bfloat16),
                    w_ref[:, pl.ds(col0, nh)],
                    preferred_element_type=jnp.float32,
                )
                dst[pl.ds(r0, SUB), :] = p.astype(jnp.bfloat16)

        own_maxes = []

        def add_stage(acc, slot, stage, r_lo, r_hi, track):
            for r0 in range(r_lo, r_hi, SUB):
                sl = pl.ds(r0, SUB)
                v = (
                    acc[slot, sl, :].astype(jnp.float32)
                    + stage[sl, :].astype(jnp.float32)
                )
                acc[slot, sl, :] = v.astype(jnp.bfloat16)
                if track:
                    own_maxes.append(jnp.max(jnp.abs(v)))

        def rs_rdma(acc, src_slot, dst_slot, s_sems, r_sems, s, sub,
                    tgt):
            rows = pl.ds(sub * QTR, QTR)
            return pltpu.make_async_remote_copy(
                src_ref=acc.at[src_slot, rows, :],
                dst_ref=acc.at[dst_slot, rows, :],
                send_sem=s_sems.at[N_RS_SUB * s + sub],
                recv_sem=r_sems.at[N_RS_SUB * s + sub],
                device_id=(tgt,),
                device_id_type=pl.DeviceIdType.MESH,
            )

        c_own_cw = lax.rem(my + 1, N_DEV)
        c_own_ccw = lax.rem(my + N_DEV - 1, N_DEV)

        pending = {}
        for sub in range(N_RS_SUB):
            r_lo, r_hi = sub * QTR, (sub + 1) * QTR
            gemm_into(acc_cw.at[1], my, 0, r_lo, r_hi)
            r = rs_rdma(acc_cw, 1, 0, send_cw, recv_cw, 0, sub, right)
            r.start()
            pending["cw", sub] = r
            gemm_into(acc_ccw.at[1], my, nh, r_lo, r_hi)
            r = rs_rdma(acc_ccw, 1, 0, send_ccw, recv_ccw, 0, sub, left)
            r.start()
            pending["ccw", sub] = r

        for s in range(N_DEV - 1):
            src_slot = (s + 1) % 2
            dst_slot = s % 2
            c_recv_cw = lax.rem(my - s - 1 + N_DEV, N_DEV)
            c_recv_ccw = lax.rem(my + s + 1, N_DEV)
            gemm_into(stage_cw, c_recv_cw, 0, 0, ch)
            gemm_into(stage_ccw, c_recv_ccw, nh, 0, ch)
            track = s == N_DEV - 2
            for sub in range(N_RS_SUB):
                r_lo, r_hi = sub * QTR, (sub + 1) * QTR
                pending["cw", sub].wait()
                if s < N_DEV - 2:
                    pl.semaphore_signal(
                        credit_cw, inc=1,
                        device_id=(left,),
                        device_id_type=pl.DeviceIdType.MESH,
                    )
                add_stage(acc_cw, dst_slot, stage_cw, r_lo, r_hi, track)
                if s < N_DEV - 2:
                    pl.semaphore_wait(credit_cw, 1)
                    r = rs_rdma(acc_cw, dst_slot, src_slot, send_cw,
                                recv_cw, s + 1, sub, right)
                    r.start()
                    pending["cw", sub] = r
                pending["ccw", sub].wait()
                if s < N_DEV - 2:
                    pl.semaphore_signal(
                        credit_ccw, inc=1,
                        device_id=(right,),
                        device_id_type=pl.DeviceIdType.MESH,
                    )
                add_stage(acc_ccw, dst_slot, stage_ccw, r_lo, r_hi,
                          track)
                if s < N_DEV - 2:
                    pl.semaphore_wait(credit_ccw, 1)
                    r = rs_rdma(acc_ccw, dst_slot, src_slot, send_ccw,
                                recv_ccw, s + 1, sub, left)
                    r.start()
                    pending["ccw", sub] = r

        amax_loc = functools.reduce(jnp.maximum, own_maxes)

        amax_ref[my] = jnp.full((8, 128), amax_loc, dtype=jnp.float32)
        amax_rdmas = []
        for t in range(N_DEV - 1):
            tgt = lax.rem(my + 1 + t, N_DEV)
            rdma = pltpu.make_async_remote_copy(
                src_ref=amax_ref.at[my],
                dst_ref=amax_ref.at[my],
                send_sem=amax_send_sems.at[t],
                recv_sem=amax_recv_sems.at[t],
                device_id=(tgt,),
                device_id_type=pl.DeviceIdType.MESH,
            )
            rdma.start()
            amax_rdmas.append(rdma)
        for rdma in amax_rdmas:
            rdma.wait()
        amax = jnp.max(amax_ref[...])

        scale = amax / 127.0
        inv_scale = 1.0 / scale

        def quant_rows(acc, c_own, col0, r_lo, r_hi):
            for r0 in range(r_lo, r_hi, SUB):
                yq = jnp.clip(
                    jnp.round(
                        acc[0, pl.ds(r0, SUB), :].astype(jnp.float32)
                        * inv_scale
                    ),
                    -127.0, 127.0,
                )
                q_ref[pl.ds(c_own * ch + r0, SUB), pl.ds(col0, nh)] = (
                    yq.astype(jnp.int8)
                )

        deq_pending = {}

        def dequant_rows(c, col0, r_lo, r_hi):
            for r0 in range(r_lo, r_hi, SUB):
                slot = (r0 // SUB) % 2
                if slot in deq_pending:
                    deq_pending.pop(slot).wait()
                sl = pl.ds(c * ch + r0, SUB)
                cs = pl.ds(col0, nh)
                deq_tiles[slot] = (
                    q_ref[sl, cs].astype(jnp.float32) * scale
                ).astype(jnp.bfloat16)
                copy = pltpu.make_async_copy(
                    deq_tiles.at[slot],
                    out_ref.at[sl, cs],
                    deq_sems.at[slot],
                )
                copy.start()
                deq_pending[slot] = copy

        def ag_rdma(c, col0, s_sems, r_sems, t, sub, tgt):
            rows = pl.ds(c * ch + sub * HALF, HALF)
            return pltpu.make_async_remote_copy(
                src_ref=q_ref.at[rows, pl.ds(col0, nh)],
                dst_ref=q_ref.at[rows, pl.ds(col0, nh)],
                send_sem=s_sems.at[4 * (N_DEV - 1) + 2 * t + sub],
                recv_sem=r_sems.at[4 * (N_DEV - 1) + 2 * t + sub],
                device_id=(tgt,),
                device_id_type=pl.DeviceIdType.MESH,
            )

        ag_pend = {}
        for sub in (0, 1):
            quant_rows(acc_cw, c_own_cw, 0, sub * HALF, (sub + 1) * HALF)
            r = ag_rdma(c_own_cw, 0, send_cw, recv_cw, 0, sub, right)
            r.start()
            ag_pend["cw", sub] = r
            quant_rows(acc_ccw, c_own_ccw, nh,
                       sub * HALF, (sub + 1) * HALF)
            r = ag_rdma(c_own_ccw, nh, send_ccw, recv_ccw, 0, sub, left)
            r.start()
            ag_pend["ccw", sub] = r
        dequant_rows(c_own_cw, 0, 0, ch)
        dequant_rows(c_own_ccw, nh, 0, ch)

        for t in range(N_DEV - 1):
            c_rcw = lax.rem(my - t + N_DEV, N_DEV)
            c_rccw = lax.rem(my + t, N_DEV)
            for sub in (0, 1):
                ag_pend["cw", sub].wait()
                if t < N_DEV - 2:
                    r = ag_rdma(c_rcw, 0, send_cw, recv_cw, t + 1, sub,
                                right)
                    r.start()
                    ag_pend["cw", sub] = r
                ag_pend["ccw", sub].wait()
                if t < N_DEV - 2:
                    r = ag_rdma(c_rccw, nh, send_ccw, recv_ccw, t + 1,
                                sub, left)
                    r.start()
                    ag_pend["ccw", sub] = r
                if t == N_DEV - 2:
                    dequant_rows(c_rcw, 0, sub * HALF, (sub + 1) * HALF)
                    dequant_rows(c_rccw, nh,
                                 sub * HALF, (sub + 1) * HALF)
            if t < N_DEV - 2:
                dequant_rows(c_rcw, 0, 0, ch)
                dequant_rows(c_rccw, nh, 0, ch)

        for copy in deq_pending.values():
            copy.wait()

    n_sems = 4 * (N_DEV - 1) + 2 * (N_DEV - 1)
    return pl.pallas_call(
        body,
        out_shape=jax.ShapeDtypeStruct((m, n), jnp.bfloat16),
        in_specs=[
            pl.BlockSpec(memory_space=pltpu.VMEM),
            pl.BlockSpec(memory_space=pltpu.VMEM),
        ],
        out_specs=pl.BlockSpec(memory_space=pl.ANY),
        scratch_shapes=[
            pltpu.VMEM((2, ch, n // 2), jnp.bfloat16),
            pltpu.VMEM((2, ch, n // 2), jnp.bfloat16),
            pltpu.VMEM((m, n), jnp.int8),
            pltpu.VMEM((ch, n // 2), jnp.bfloat16),
            pltpu.VMEM((ch, n // 2), jnp.bfloat16),
            pltpu.VMEM((2, 256, n // 2), jnp.bfloat16),
            pltpu.VMEM((N_DEV, 8, 128), jnp.float32),
            pltpu.SemaphoreType.DMA((n_sems,)),
            pltpu.SemaphoreType.DMA((n_sems,)),
            pltpu.SemaphoreType.DMA((n_sems,)),
            pltpu.SemaphoreType.DMA((n_sems,)),
            pltpu.SemaphoreType.DMA((N_DEV - 1,)),
            pltpu.SemaphoreType.DMA((N_DEV - 1,)),
            pltpu.SemaphoreType.DMA((2,)),
            pltpu.SemaphoreType.REGULAR,
            pltpu.SemaphoreType.REGULAR,
        ],
        compiler_params=pltpu.CompilerParams(
            collective_id=0,
            vmem_limit_bytes=40 * 1024 * 1024,
        ),
    )(x, w_mat)
